# baseline (speedup 1.0000x reference)
import functools

import jax
import jax.numpy as jnp
from jax import lax
from jax.experimental import pallas as pl
from jax.experimental.pallas import tpu as pltpu
from jax.experimental.pallas import tpu_sc as plsc

_LANES = 16
_CHUNK = 128
_KDEPTH = 8


def _sc_feature_gather(x, feature_mask):
    B, S, F = x.shape
    K = feature_mask.shape[0]
    R = B * S

    info = plsc.get_sparse_core_info()
    NC, NS = info.num_cores, info.num_subcores
    NW = NC * NS
    RPW = R // NW
    rows_per_chunk = _CHUNK // K
    nchunk = RPW // rows_per_chunk
    ngroup = nchunk // _KDEPTH

    x_flat = x.reshape(R * F)

    @functools.partial(
        pl.kernel,
        out_type=jax.ShapeDtypeStruct((R * K,), jnp.float32),
        mesh=plsc.VectorSubcoreMesh(core_axis_name="c", subcore_axis_name="s"),
        scratch_types=[
            pltpu.VMEM((K,), jnp.int32),
            pltpu.VMEM((nchunk, _CHUNK), jnp.int32),
            pltpu.VMEM((RPW * K,), jnp.float32),
            pltpu.SemaphoreType.DMA,
        ],
    )
    def sc_kernel(x_hbm, mask_hbm, out_hbm, mask_v, idx_v, out_v, sem):
        wid = lax.axis_index("s") * NC + lax.axis_index("c")
        row0 = wid * RPW
        pltpu.sync_copy(mask_hbm, mask_v)

        mvecs = [mask_v[pl.ds(j * _LANES, _LANES)] for j in range(K // _LANES)]

        def group_body(g, carry):
            copies = []
            for i in range(_KDEPTH):
                c = g * _KDEPTH + i
                for rr in range(rows_per_chunk):
                    base = (row0 + c * rows_per_chunk + rr) * F
                    for j in range(K // _LANES):
                        idx_v[c, pl.ds(rr * K + j * _LANES, _LANES)] = (
                            mvecs[j] + base
                        )
                copies.append(
                    pltpu.async_copy(
                        x_hbm.at[idx_v.at[c]],
                        out_v.at[pl.ds(c * _CHUNK, _CHUNK)],
                        sem,
                    )
                )
            for cp in copies:
                cp.wait()
            return carry

        lax.fori_loop(0, ngroup, group_body, 0)
        pltpu.sync_copy(out_v, out_hbm.at[pl.ds(row0 * K, RPW * K)])

    out = sc_kernel(x_flat, feature_mask)
    return out.reshape(B, S, K)


def kernel(x, feature_mask):
    return _sc_feature_gather(x, feature_mask)
